# Initial kernel scaffold; baseline (speedup 1.0000x reference)
import jax
import jax.numpy as jnp
from jax.experimental import pallas as pl


def kernel(image_features, text_features, W1, b1, W2, b2, W3, b3, emb_table, text_query):
    raise NotImplementedError("write your pallas kernel here")


# speedup vs baseline: 1.6271x; 1.6271x over previous
import jax
import jax.numpy as jnp
from jax.experimental import pallas as pl
from jax.experimental.pallas import tpu as pltpu

B = 4096
NUM_IDS = 10000
EMB_DIM = 256
BLK_B = 128
NB = B // BLK_B


def _fused_kernel(img_ref, txt_ref, w1_ref, b1_ref, w2_ref, b2_ref,
                  w3_ref, b3_ref, emb_ref, sims_ref, idx_ref, embn_ref, buf_ref):
    i = pl.program_id(0)

    @pl.when(i == 0)
    def _():
        emb = emb_ref[...]
        nrm = jnp.sqrt(jnp.sum(emb * emb, axis=1, keepdims=True))
        embn_ref[...] = emb / jnp.maximum(nrm, 1e-8)

    work = buf_ref[...]
    sims_ref[...] = work
    col = jax.lax.broadcasted_iota(jnp.int32, work.shape, 1)
    idxs = []
    for _ in range(5):
        m = jnp.max(work, axis=1, keepdims=True)
        idx = jnp.min(jnp.where(work == m, col, NUM_IDS), axis=1)
        idxs.append(idx[:, None])
        work = jnp.where(col == idx[:, None], -jnp.inf, work)
    idx_ref[...] = jnp.concatenate(idxs, axis=1)

    cat = jnp.concatenate([img_ref[...], txt_ref[...]], axis=1)
    h = jnp.dot(cat, w1_ref[...], preferred_element_type=jnp.float32)
    h = jax.nn.relu(h + b1_ref[...])
    h = jax.nn.relu(jnp.dot(h, w2_ref[...], preferred_element_type=jnp.float32) + b2_ref[...])
    proj = jnp.dot(h, w3_ref[...], preferred_element_type=jnp.float32) + b3_ref[...]
    nrm = jnp.sqrt(jnp.sum(proj * proj, axis=1, keepdims=True))
    proj_n = proj / jnp.maximum(nrm, 1e-8)
    buf_ref[...] = jax.lax.dot_general(proj_n, embn_ref[...],
                                       (((1,), (1,)), ((), ())),
                                       preferred_element_type=jnp.float32)


@jax.jit
def kernel(image_features, text_features, W1, b1, W2, b2, W3, b3, emb_table,
           text_query=0):
    di = image_features.shape[1]
    full = lambda i: (0, 0)
    feed = lambda i: (jnp.minimum(i, NB - 1), 0)
    drain = lambda i: (jnp.maximum(i - 1, 0), 0)
    sims, idx = pl.pallas_call(
        _fused_kernel,
        grid=(NB + 1,),
        in_specs=[
            pl.BlockSpec((BLK_B, di), feed),
            pl.BlockSpec((BLK_B, di), feed),
            pl.BlockSpec(W1.shape, full),
            pl.BlockSpec((1, 1024), full),
            pl.BlockSpec(W2.shape, full),
            pl.BlockSpec((1, 512), full),
            pl.BlockSpec(W3.shape, full),
            pl.BlockSpec((1, EMB_DIM), full),
            pl.BlockSpec(emb_table.shape, full),
        ],
        out_specs=[
            pl.BlockSpec((BLK_B, NUM_IDS), drain),
            pl.BlockSpec((BLK_B, 5), drain),
        ],
        out_shape=[
            jax.ShapeDtypeStruct((B, NUM_IDS), jnp.float32),
            jax.ShapeDtypeStruct((B, 5), jnp.int32),
        ],
        scratch_shapes=[
            pltpu.VMEM((NUM_IDS, EMB_DIM), jnp.float32),
            pltpu.VMEM((BLK_B, NUM_IDS), jnp.float32),
        ],
    )(image_features, text_features, W1, b1.reshape(1, -1),
      W2, b2.reshape(1, -1), W3, b3.reshape(1, -1), emb_table)
    return (sims, idx)
